# baseline (speedup 1.0000x reference)
import functools

import jax
import jax.numpy as jnp
from jax import lax
from jax.experimental import pallas as pl
from jax.experimental.pallas import tpu as pltpu
from jax.experimental.pallas import tpu_sc as plsc

D_MODEL = 32


@jax.jit
def _embed_impl(xt, weight):
    S, BT = xt.shape
    D = D_MODEL
    info = plsc.get_sparse_core_info()
    NC = info.num_cores
    NW = NC * info.num_subcores
    NBT = BT // 128
    BTW = NBT // NW
    NFT = D // 8
    NK = S // 2

    mesh = plsc.VectorSubcoreMesh(core_axis_name="c", subcore_axis_name="s")

    @functools.partial(
        pl.kernel,
        mesh=mesh,
        out_type=jax.ShapeDtypeStruct((S, NBT, 128), jnp.int32),
        scratch_types=[pltpu.VMEM((8, 128), jnp.int32)],
        compiler_params=pltpu.CompilerParams(use_tc_tiling_on_sc=True),
    )
    def untile(xt_hbm, idx_hbm, tile_v):
        wid = lax.axis_index("s") * NC + lax.axis_index("c")
        for c in range(BTW):
            bt = wid * BTW + c
            for r in range((S + 7) // 8):
                h = min(8, S - r * 8)
                pltpu.sync_copy(
                    xt_hbm.at[pl.ds(r * 8, h), pl.ds(bt * 128, 128)],
                    tile_v.at[pl.ds(0, h)],
                )
                pltpu.sync_copy(
                    tile_v.at[pl.ds(0, h)],
                    idx_hbm.at[pl.ds(r * 8, h), bt],
                )

    @functools.partial(
        pl.kernel,
        mesh=mesh,
        out_type=jax.ShapeDtypeStruct((S, NFT, NBT, 8, 128), jnp.float32),
        scratch_types=[
            pltpu.VMEM((S, BTW, 128), jnp.int32),
            pltpu.VMEM((BTW * 128, D), jnp.float32),
            pltpu.VMEM((BTW * 128, D), jnp.float32),
            pltpu.VMEM((NFT, BTW, 8, 128), jnp.float32),
            pltpu.VMEM((NFT, BTW, 8, 128), jnp.float32),
            pltpu.SemaphoreType.DMA,
            pltpu.SemaphoreType.DMA,
        ],
        compiler_params=pltpu.CompilerParams(
            use_tc_tiling_on_sc=False, needs_layout_passes=False
        ),
    )
    def gat(idx_hbm, table_hbm, out_hbm, idx_all, rows0, rows1, t0, t1,
            gsem, osem):
        wid = lax.axis_index("s") * NC + lax.axis_index("c")
        bt0 = wid * BTW
        iota = lax.iota(jnp.int32, 16)

        pltpu.sync_copy(idx_hbm.at[:, pl.ds(bt0, BTW)], idx_all)

        def fire_gathers(s, rbuf):
            for q in range(BTW):
                pltpu.async_copy(
                    table_hbm.at[idx_all.at[s, q]],
                    rbuf.at[pl.ds(q * 128, 128)],
                    gsem,
                )

        def drain_gathers(rbuf):
            for q in range(BTW):
                pltpu.make_async_copy(
                    table_hbm.at[idx_all.at[0, 0]],
                    rbuf.at[pl.ds(q * 128, 128)],
                    gsem,
                ).wait()

        def drain_wb(t):
            pltpu.make_async_copy(
                t, out_hbm.at[0, :, pl.ds(bt0, BTW)], osem
            ).wait()

        def transpose(r, t):
            def jbody(j, c):
                rowv = iota + j * 16
                bt = j >> 3
                m = j & 7
                for ft in range(NFT):
                    for fs in range(8):
                        col = jnp.full((16,), ft * 8 + fs, jnp.int32)
                        v = plsc.load_gather(r, [rowv, col])
                        t[ft, bt, fs, pl.ds(m * 16, 16)] = v
                return c
            lax.fori_loop(0, BTW * 8, jbody, 0)

        def half_step(s, r, t, first):
            drain_gathers(r)
            if not first:
                drain_wb(t)
            transpose(r, t)
            pltpu.async_copy(t, out_hbm.at[s, :, pl.ds(bt0, BTW)], osem)

        fire_gathers(0, rows0)
        fire_gathers(1, rows1)

        def kbody(k, carry):
            s0 = k * 2

            @pl.when(k > 0)
            def _():
                drain_wb(t0)
                drain_wb(t1)

            drain_gathers(rows0)
            transpose(rows0, t0)
            pltpu.async_copy(t0, out_hbm.at[s0, :, pl.ds(bt0, BTW)], osem)

            @pl.when(k < NK - 1)
            def _():
                fire_gathers(s0 + 2, rows0)

            drain_gathers(rows1)
            transpose(rows1, t1)
            pltpu.async_copy(
                t1, out_hbm.at[s0 + 1, :, pl.ds(bt0, BTW)], osem
            )

            @pl.when(k < NK - 1)
            def _():
                fire_gathers(s0 + 3, rows1)

            return carry

        lax.fori_loop(0, NK, kbody, 0)
        drain_wb(t0)
        drain_wb(t1)

    idx3 = untile(xt)
    out_lin = gat(idx3, weight)
    return out_lin


def kernel(x, weight):
    out_lin = _embed_impl(x.T, weight)
    S = x.shape[1]
    return (
        out_lin.transpose(2, 4, 0, 1, 3)
        .reshape(x.shape[0], S, D_MODEL)
    )
